# Initial kernel scaffold; baseline (speedup 1.0000x reference)
import jax
import jax.numpy as jnp
from jax.experimental import pallas as pl


def kernel(x, noise, Wg, Wnoise):
    raise NotImplementedError("write your pallas kernel here")


# speedup vs baseline: 5.2297x; 5.2297x over previous
import functools

import jax
import jax.numpy as jnp
from jax.experimental import pallas as pl

DIM = 4096
E = 64
T_BLOCK = 512
NEG_INF = float("-inf")


def _router_block(x_ref, w_ref, noise_ref, out_ref):
    logits = jnp.dot(x_ref[...], w_ref[...], preferred_element_type=jnp.float32)
    clean = logits[:, :E]
    std = jax.nn.softplus(logits[:, E:]) + 1e-2
    v = clean + noise_ref[...] * std
    work = v
    for _ in range(7):
        m = jnp.max(work, axis=-1, keepdims=True)
        work = jnp.where(work == m, NEG_INF, work)
    thresh = jnp.max(work, axis=-1, keepdims=True)
    rowmax = jnp.max(v, axis=-1, keepdims=True)
    e = jnp.where(v >= thresh, jnp.exp(v - rowmax), 0.0)
    out_ref[...] = e / jnp.sum(e, axis=-1, keepdims=True)


@jax.jit
def kernel(x, noise, Wg, Wnoise):
    t = x.shape[0]
    w = jnp.concatenate([Wg, Wnoise], axis=1)
    grid = (t // T_BLOCK,)
    return pl.pallas_call(
        _router_block,
        grid=grid,
        in_specs=[
            pl.BlockSpec((T_BLOCK, DIM), lambda i: (i, 0)),
            pl.BlockSpec((DIM, 2 * E), lambda i: (0, 0)),
            pl.BlockSpec((T_BLOCK, E), lambda i: (i, 0)),
        ],
        out_specs=pl.BlockSpec((T_BLOCK, E), lambda i: (i, 0)),
        out_shape=jax.ShapeDtypeStruct((t, E), jnp.float32),
    )(x, w, noise)


# speedup vs baseline: 5.4241x; 1.0372x over previous
import functools

import jax
import jax.numpy as jnp
from jax.experimental import pallas as pl

DIM = 4096
E = 64
T_BLOCK = 1024
NEG_INF = float("-inf")


def _router_block(x_ref, w_ref, noise_ref, out_ref):
    logits = jnp.dot(x_ref[...], w_ref[...], preferred_element_type=jnp.float32)
    clean = logits[:, :E]
    std = jax.nn.softplus(logits[:, E:]) + 1e-2
    v = clean + noise_ref[...] * std
    work = v
    for _ in range(7):
        m = jnp.max(work, axis=-1, keepdims=True)
        work = jnp.where(work == m, NEG_INF, work)
    thresh = jnp.max(work, axis=-1, keepdims=True)
    rowmax = jnp.max(v, axis=-1, keepdims=True)
    e = jnp.where(v >= thresh, jnp.exp(v - rowmax), 0.0)
    out_ref[...] = e / jnp.sum(e, axis=-1, keepdims=True)


@jax.jit
def kernel(x, noise, Wg, Wnoise):
    t = x.shape[0]
    w = jnp.concatenate([Wg, Wnoise], axis=1)
    grid = (t // T_BLOCK,)
    return pl.pallas_call(
        _router_block,
        grid=grid,
        in_specs=[
            pl.BlockSpec((T_BLOCK, DIM), lambda i: (i, 0)),
            pl.BlockSpec((DIM, 2 * E), lambda i: (0, 0)),
            pl.BlockSpec((T_BLOCK, E), lambda i: (i, 0)),
        ],
        out_specs=pl.BlockSpec((T_BLOCK, E), lambda i: (i, 0)),
        out_shape=jax.ShapeDtypeStruct((t, E), jnp.float32),
    )(x, w, noise)
